# Initial kernel scaffold; baseline (speedup 1.0000x reference)
import jax
import jax.numpy as jnp
from jax.experimental import pallas as pl


def kernel(x, embedding, proj_W, proj_b, W1, a_src1, a_dst1, b1, W2, a_src2, a_dst2, b2):
    raise NotImplementedError("write your pallas kernel here")


# speedup vs baseline: 1.8383x; 1.8383x over previous
import functools

import jax
import jax.numpy as jnp
from jax.experimental import pallas as pl
from jax.experimental.pallas import tpu as pltpu

N = 1024
IN_DIM = 256
HID = 128
HEADS = 4
FH = HID // HEADS
NEG = -1e9


def _gat_layer(h_in, w_r, a_src, a_dst, bias, mask):
    hv = jnp.dot(h_in, w_r, preferred_element_type=jnp.float32)
    outs = []
    for hd in range(HEADS):
        hh = hv[:, hd * FH:(hd + 1) * FH]
        a_s = a_src[hd:hd + 1, :]
        a_d = a_dst[hd:hd + 1, :]
        s_row = jax.lax.dot_general(
            a_s, hh, (((1,), (1,)), ((), ())),
            preferred_element_type=jnp.float32)
        d_col = jax.lax.dot_general(
            hh, a_d, (((1,), (1,)), ((), ())),
            preferred_element_type=jnp.float32)
        z0 = d_col + s_row
        lr = jnp.where(z0 >= 0, z0, 0.2 * z0)
        z = jnp.where(mask != 0, lr, NEG)
        m = jnp.max(z, axis=1, keepdims=True)
        p = jnp.exp(z - m)
        den = jnp.sum(p, axis=1, keepdims=True)
        o = jnp.dot(p, hh, preferred_element_type=jnp.float32) / den
        outs.append(o)
    return jnp.concatenate(outs, axis=1) + bias


def _body(x_ref, emb_ref, pw_ref, pb_ref, w1_ref, as1_ref, ad1_ref, b1_ref,
          w2_ref, as2_ref, ad2_ref, b2_ref, out_ref, mask_ref):
    b = pl.program_id(0)

    @pl.when(b == 0)
    def _():
        emb = emb_ref[:]
        ssq = jnp.sum(emb * emb, axis=1, keepdims=True)
        nrm = emb * jax.lax.rsqrt(jnp.maximum(ssq, 1e-12))
        adj = jax.lax.dot_general(
            nrm, nrm, (((1,), (1,)), ((), ())),
            preferred_element_type=jnp.float32)
        mask_ref[:] = (adj > 0.5).astype(jnp.float32)

    mask = mask_ref[:]
    x = x_ref[0]
    h0 = jnp.dot(x, pw_ref[:], preferred_element_type=jnp.float32) + pb_ref[:]
    h1 = _gat_layer(h0, w1_ref[:], as1_ref[:], ad1_ref[:], b1_ref[:], mask)
    h1 = jax.nn.gelu(h1)
    h2 = _gat_layer(h1, w2_ref[:], as2_ref[:], ad2_ref[:], b2_ref[:], mask)
    out_ref[0] = h2


@functools.partial(jax.jit, static_argnames=())
def kernel(x, embedding, proj_W, proj_b, W1, a_src1, a_dst1, b1,
           W2, a_src2, a_dst2, b2):
    B = x.shape[0]
    w1r = W1.reshape(HID, HID)
    w2r = W2.reshape(HID, HID)
    pb = proj_b.reshape(1, HID)
    b1r = b1.reshape(1, HID)
    b2r = b2.reshape(1, HID)

    full = lambda shape: pl.BlockSpec(shape, lambda b: (0,) * len(shape))
    out = pl.pallas_call(
        _body,
        grid=(B,),
        in_specs=[
            pl.BlockSpec((1, N, IN_DIM), lambda b: (b, 0, 0)),
            full((N, HID)),
            full((IN_DIM, HID)),
            full((1, HID)),
            full((HID, HID)),
            full((HEADS, FH)),
            full((HEADS, FH)),
            full((1, HID)),
            full((HID, HID)),
            full((HEADS, FH)),
            full((HEADS, FH)),
            full((1, HID)),
        ],
        out_specs=pl.BlockSpec((1, N, HID), lambda b: (b, 0, 0)),
        out_shape=jax.ShapeDtypeStruct((B, N, HID), jnp.float32),
        scratch_shapes=[pltpu.VMEM((N, N), jnp.float32)],
        compiler_params=pltpu.CompilerParams(
            dimension_semantics=("arbitrary",)),
    )(x, embedding, proj_W, pb, w1r, a_src1, a_dst1, b1r,
      w2r, a_src2, a_dst2, b2r)
    return out


# speedup vs baseline: 2.2031x; 1.1985x over previous
import functools

import jax
import jax.numpy as jnp
from jax.experimental import pallas as pl
from jax.experimental.pallas import tpu as pltpu

N = 1024
IN_DIM = 256
HID = 128
HEADS = 4
FH = HID // HEADS
NEG = -1e9


def _gat_layer(h_in, w_r, a_src, a_dst, bias, mask):
    hv = jnp.dot(h_in, w_r, preferred_element_type=jnp.float32)
    outs = []
    for hd in range(HEADS):
        hh = hv[:, hd * FH:(hd + 1) * FH]
        a_s = a_src[hd:hd + 1, :]
        a_d = a_dst[hd:hd + 1, :]
        s_row = jax.lax.dot_general(
            a_s, hh, (((1,), (1,)), ((), ())),
            preferred_element_type=jnp.float32)
        d_col = jax.lax.dot_general(
            hh, a_d, (((1,), (1,)), ((), ())),
            preferred_element_type=jnp.float32)
        sm = jnp.max(jnp.where(mask != 0, s_row, NEG), axis=1, keepdims=True)
        zc = d_col + sm
        c = jnp.where(zc >= 0, zc, 0.2 * zc)
        u1 = jnp.exp(d_col - c)
        u2 = jnp.exp(0.2 * d_col - c)
        v1 = jnp.exp(s_row)
        v2 = jnp.exp(0.2 * s_row)
        w = jnp.maximum(u1 * v1, u2 * v2)
        w = jnp.minimum(w, 3e38)
        p = mask * w
        hh_aug = jnp.concatenate(
            [hh, jnp.ones((N, 1), jnp.float32)], axis=1)
        res = jnp.dot(p, hh_aug, preferred_element_type=jnp.float32)
        den = res[:, FH:FH + 1]
        mh = jnp.mean(hh, axis=0, keepdims=True)
        o = jnp.where(den > 0, res[:, :FH] / den, mh)
        outs.append(o)
    return jnp.concatenate(outs, axis=1) + bias


def _body(x_ref, emb_ref, pw_ref, pb_ref, w1_ref, as1_ref, ad1_ref, b1_ref,
          w2_ref, as2_ref, ad2_ref, b2_ref, out_ref, mask_ref):
    b = pl.program_id(0)

    @pl.when(b == 0)
    def _():
        emb = emb_ref[:]
        ssq = jnp.sum(emb * emb, axis=1, keepdims=True)
        nrm = emb * jax.lax.rsqrt(jnp.maximum(ssq, 1e-12))
        adj = jax.lax.dot_general(
            nrm, nrm, (((1,), (1,)), ((), ())),
            preferred_element_type=jnp.float32)
        mask_ref[:] = (adj > 0.5).astype(jnp.float32)

    mask = mask_ref[:]
    x = x_ref[0]
    h0 = jnp.dot(x, pw_ref[:], preferred_element_type=jnp.float32) + pb_ref[:]
    h1 = _gat_layer(h0, w1_ref[:], as1_ref[:], ad1_ref[:], b1_ref[:], mask)
    h1 = jax.nn.gelu(h1)
    h2 = _gat_layer(h1, w2_ref[:], as2_ref[:], ad2_ref[:], b2_ref[:], mask)
    out_ref[0] = h2


@functools.partial(jax.jit, static_argnames=())
def kernel(x, embedding, proj_W, proj_b, W1, a_src1, a_dst1, b1,
           W2, a_src2, a_dst2, b2):
    B = x.shape[0]
    w1r = W1.reshape(HID, HID)
    w2r = W2.reshape(HID, HID)
    pb = proj_b.reshape(1, HID)
    b1r = b1.reshape(1, HID)
    b2r = b2.reshape(1, HID)

    full = lambda shape: pl.BlockSpec(shape, lambda b: (0,) * len(shape))
    out = pl.pallas_call(
        _body,
        grid=(B,),
        in_specs=[
            pl.BlockSpec((1, N, IN_DIM), lambda b: (b, 0, 0)),
            full((N, HID)),
            full((IN_DIM, HID)),
            full((1, HID)),
            full((HID, HID)),
            full((HEADS, FH)),
            full((HEADS, FH)),
            full((1, HID)),
            full((HID, HID)),
            full((HEADS, FH)),
            full((HEADS, FH)),
            full((1, HID)),
        ],
        out_specs=pl.BlockSpec((1, N, HID), lambda b: (b, 0, 0)),
        out_shape=jax.ShapeDtypeStruct((B, N, HID), jnp.float32),
        scratch_shapes=[pltpu.VMEM((N, N), jnp.float32)],
        compiler_params=pltpu.CompilerParams(
            dimension_semantics=("arbitrary",)),
    )(x, embedding, proj_W, pb, w1r, a_src1, a_dst1, b1r,
      w2r, a_src2, a_dst2, b2r)
    return out


# speedup vs baseline: 2.3455x; 1.0647x over previous
import functools

import jax
import jax.numpy as jnp
from jax.experimental import pallas as pl
from jax.experimental.pallas import tpu as pltpu

N = 1024
IN_DIM = 256
HID = 128
HEADS = 4
FH = HID // HEADS
NEG = -1e9


def _gat_layer(h_in, w_r, a_src, a_dst, bias, mask, madd):
    hv = jnp.dot(h_in, w_r, preferred_element_type=jnp.float32)
    outs = []
    for hd in range(HEADS):
        hh = hv[:, hd * FH:(hd + 1) * FH]
        a_s = a_src[hd:hd + 1, :]
        a_d = a_dst[hd:hd + 1, :]
        s_row = jax.lax.dot_general(
            a_s, hh, (((1,), (1,)), ((), ())),
            preferred_element_type=jnp.float32)
        d_col = jax.lax.dot_general(
            hh, a_d, (((1,), (1,)), ((), ())),
            preferred_element_type=jnp.float32)
        sm = jnp.max(s_row + madd, axis=1, keepdims=True)
        zc = d_col + sm
        c = jnp.where(zc >= 0, zc, 0.2 * zc)
        u1 = jnp.exp(d_col - c)
        u2 = jnp.exp(0.2 * d_col - c)
        v1 = jnp.exp(s_row)
        v2 = jnp.exp(0.2 * s_row)
        w = jnp.maximum(u1 * v1, u2 * v2)
        w = jnp.minimum(w, 3e38)
        p = mask * w
        hh_aug = jnp.concatenate(
            [hh, jnp.ones((N, 1), jnp.float32)], axis=1)
        res = jnp.dot(p, hh_aug, preferred_element_type=jnp.float32)
        den = res[:, FH:FH + 1]
        mh = jnp.mean(hh, axis=0, keepdims=True)
        o = jnp.where(den > 0, res[:, :FH] / den, mh)
        outs.append(o)
    return jnp.concatenate(outs, axis=1) + bias


def _body(x_ref, emb_ref, pw_ref, pb_ref, w1_ref, as1_ref, ad1_ref, b1_ref,
          w2_ref, as2_ref, ad2_ref, b2_ref, out_ref, mask_ref, madd_ref):
    b = pl.program_id(0)

    @pl.when(b == 0)
    def _():
        emb = emb_ref[:]
        ssq = jnp.sum(emb * emb, axis=1, keepdims=True)
        nrm = emb * jax.lax.rsqrt(jnp.maximum(ssq, 1e-12))
        adj = jax.lax.dot_general(
            nrm, nrm, (((1,), (1,)), ((), ())),
            preferred_element_type=jnp.float32)
        edge = adj > 0.5
        mask_ref[:] = edge.astype(jnp.float32)
        madd_ref[:] = jnp.where(edge, 0.0, NEG)

    mask = mask_ref[:]
    madd = madd_ref[:]
    x = x_ref[0]
    h0 = jnp.dot(x, pw_ref[:], preferred_element_type=jnp.float32) + pb_ref[:]
    h1 = _gat_layer(h0, w1_ref[:], as1_ref[:], ad1_ref[:], b1_ref[:],
                    mask, madd)
    h1 = jax.nn.gelu(h1)
    h2 = _gat_layer(h1, w2_ref[:], as2_ref[:], ad2_ref[:], b2_ref[:],
                    mask, madd)
    out_ref[0] = h2


@functools.partial(jax.jit, static_argnames=())
def kernel(x, embedding, proj_W, proj_b, W1, a_src1, a_dst1, b1,
           W2, a_src2, a_dst2, b2):
    B = x.shape[0]
    w1r = W1.reshape(HID, HID)
    w2r = W2.reshape(HID, HID)
    pb = proj_b.reshape(1, HID)
    b1r = b1.reshape(1, HID)
    b2r = b2.reshape(1, HID)

    full = lambda shape: pl.BlockSpec(shape, lambda b: (0,) * len(shape))
    out = pl.pallas_call(
        _body,
        grid=(B,),
        in_specs=[
            pl.BlockSpec((1, N, IN_DIM), lambda b: (b, 0, 0)),
            full((N, HID)),
            full((IN_DIM, HID)),
            full((1, HID)),
            full((HID, HID)),
            full((HEADS, FH)),
            full((HEADS, FH)),
            full((1, HID)),
            full((HID, HID)),
            full((HEADS, FH)),
            full((HEADS, FH)),
            full((1, HID)),
        ],
        out_specs=pl.BlockSpec((1, N, HID), lambda b: (b, 0, 0)),
        out_shape=jax.ShapeDtypeStruct((B, N, HID), jnp.float32),
        scratch_shapes=[pltpu.VMEM((N, N), jnp.float32),
                        pltpu.VMEM((N, N), jnp.float32)],
        compiler_params=pltpu.CompilerParams(
            dimension_semantics=("arbitrary",)),
    )(x, embedding, proj_W, pb, w1r, a_src1, a_dst1, b1r,
      w2r, a_src2, a_dst2, b2r)
    return out
